# Initial kernel scaffold; baseline (speedup 1.0000x reference)
import jax
import jax.numpy as jnp
from jax.experimental import pallas as pl


def kernel(x_gene, x_disease, edge_index_A, edge_index_B, label_A, label_B, params):
    raise NotImplementedError("write your pallas kernel here")


# speedup vs baseline: 4.6260x; 4.6260x over previous
import functools

import jax
import jax.numpy as jnp
from jax import lax
from jax.experimental import pallas as pl
from jax.experimental.pallas import tpu as pltpu
from jax.experimental.pallas import tpu_sc as plsc

HID = 32
N = 10000
NPAD = 10112
RPT = NPAD // 16
NP4 = NPAD // 4
NR4 = N // 4
E = 320000
KC = 128
NW = 32
CPW = 79
EPW = CPW * KC
EPAD = NW * EPW
L = 100000
LCPW = 25
LPW = LCPW * KC
LPAD = NW * LPW

_SDS = jax.ShapeDtypeStruct


def _mesh():
    return plsc.VectorSubcoreMesh(core_axis_name="c", subcore_axis_name="s")


def _dot(a, b):
    return jnp.dot(a, b, preferred_element_type=jnp.float32,
                   precision=lax.Precision.HIGHEST)


def _fold_src(ws, wu):
    return _dot(ws, wu[HID:, :])


def _fold_dst(wd, wu):
    return _dot(wd, wu[:HID, :])


def _fold_bias(bs, bd, wu, bu):
    return (_dot(bd.reshape(1, HID), wu[:HID, :])
            + _dot(bs.reshape(1, HID), wu[HID:, :]) + bu.reshape(1, HID))


def _bd4(w):
    z = jnp.zeros(w.shape, jnp.float32)
    return jnp.concatenate([
        jnp.concatenate([w, z, z, z], axis=1),
        jnp.concatenate([z, w, z, z], axis=1),
        jnp.concatenate([z, z, w, z], axis=1),
        jnp.concatenate([z, z, z, w], axis=1)], axis=0)


def _fold4(v):
    return v[0:HID] + v[HID:2 * HID] + v[2 * HID:3 * HID] + v[3 * HID:]


def _rep4(v):
    return jnp.concatenate([v, v, v, v])


def _bn_act_packed(h, gamma, beta, leaky):
    hv = h[0:NR4, :]
    mu = _rep4(_fold4(jnp.sum(hv, axis=0)) / N)
    e2 = _rep4(_fold4(jnp.sum(hv * hv, axis=0)) / N)
    var = e2 - mu * mu
    out = (_rep4(gamma)[None, :] * (h - mu[None, :])
           * lax.rsqrt(var[None, :] + 1e-5) + _rep4(beta)[None, :])
    if leaky:
        out = jnp.where(out >= 0, out, 0.01 * out)
    return out


def _t1_body(xg, xd, wsa, bsa, wda, bda, wua, bua,
             wsb, bsb, wdb, bdb, wub, bub,
             srcA, dstA, srcB, dstB):
    srcA[...] = _dot(xg[...], _bd4(_fold_src(wsa[...], wua[...])))
    dstA[...] = _dot(xd[...], _bd4(_fold_dst(wda[...], wua[...]))) + _rep4(
        _fold_bias(bsa[...], bda[...], wua[...], bua[...])[0])[None, :]
    srcB[...] = _dot(xd[...], _bd4(_fold_src(wsb[...], wub[...])))
    dstB[...] = _dot(xg[...], _bd4(_fold_dst(wdb[...], wub[...]))) + _rep4(
        _fold_bias(bsb[...], bdb[...], wub[...], bub[...])[0])[None, :]


def _t2_body(dstA, dstB, agA, cnA, agB, cnB,
             g1g, b1g, g1d, b1d,
             wsa, bsa, wda, bda, wua, bua,
             wsb, bsb, wdb, bdb, wub, bub,
             srcA2, dstA2, srcB2, dstB2):
    sumA = agA[0:NP4, :] + agA[NP4:, :]
    cdis = jnp.maximum(cnA[0:NP4, :] + cnA[NP4:, :], 1.0)
    h_dis = _bn_act_packed(dstA[...] + sumA / cdis, g1d[...], b1d[...], True)
    sumB = agB[0:NP4, :] + agB[NP4:, :]
    cgen = jnp.maximum(cnB[0:NP4, :] + cnB[NP4:, :], 1.0)
    h_gene = _bn_act_packed(dstB[...] + sumB / cgen, g1g[...], b1g[...], True)
    srcA2[...] = _dot(h_gene, _bd4(_fold_src(wsa[...], wua[...])))
    dstA2[...] = _dot(h_dis, _bd4(_fold_dst(wda[...], wua[...]))) + _rep4(
        _fold_bias(bsa[...], bda[...], wua[...], bua[...])[0])[None, :]
    srcB2[...] = _dot(h_dis, _bd4(_fold_src(wsb[...], wub[...])))
    dstB2[...] = _dot(h_gene, _bd4(_fold_dst(wdb[...], wub[...]))) + _rep4(
        _fold_bias(bsb[...], bdb[...], wub[...], bub[...])[0])[None, :]


def _t3_body(dstA2, dstB2, ag2A, ag2B, cnA, cnB,
             g2g, b2g, g2d, b2d, h2g, h2d):
    sumA = ag2A[0:NP4, :] + ag2A[NP4:, :]
    cdis = jnp.maximum(cnA[0:NP4, :] + cnA[NP4:, :], 1.0)
    h2d[...] = _bn_act_packed(dstA2[...] + sumA / cdis, g2d[...], b2d[...], False)
    sumB = ag2B[0:NP4, :] + ag2B[NP4:, :]
    cgen = jnp.maximum(cnB[0:NP4, :] + cnB[NP4:, :], 1.0)
    h2g[...] = _bn_act_packed(dstB2[...] + sumB / cgen, g2g[...], b2g[...], False)


_SC_PARAMS = dict(use_tc_tiling_on_sc=False, needs_layout_passes=False)


def _seg1_kernel():
    @functools.partial(
        pl.kernel, mesh=_mesh(),
        compiler_params=pltpu.CompilerParams(**_SC_PARAMS),
        out_type=[_SDS((2 * NPAD, HID), jnp.float32)] * 4,
        scratch_types=[pltpu.VMEM((KC,), jnp.int32),
                       pltpu.VMEM((KC,), jnp.int32),
                       pltpu.VMEM((KC, HID), jnp.float32),
                       pltpu.VMEM((KC, HID), jnp.float32),
                       pltpu.VMEM_SHARED((NPAD, HID), jnp.float32),
                       pltpu.VMEM_SHARED((NPAD, HID), jnp.float32),
                       pltpu.VMEM_SHARED((NPAD, HID), jnp.float32),
                       pltpu.VMEM_SHARED((NPAD, HID), jnp.float32)],
    )
    def seg(srca, ea0, ea1, srcb, eb0, eb1, zrow, ones_in,
            aggrA, cntA, aggrB, cntB,
            sidx, didx, rows, onesv, accA, accB, ccA, ccB):
        core = lax.axis_index("c")
        sub = lax.axis_index("s")
        wid = sub * 2 + core
        r0 = sub * RPT
        pltpu.sync_copy(zrow.at[pl.ds(r0, RPT)], accA.at[pl.ds(r0, RPT)])
        pltpu.sync_copy(zrow.at[pl.ds(r0, RPT)], accB.at[pl.ds(r0, RPT)])
        pltpu.sync_copy(zrow.at[pl.ds(r0, RPT)], ccA.at[pl.ds(r0, RPT)])
        pltpu.sync_copy(zrow.at[pl.ds(r0, RPT)], ccB.at[pl.ds(r0, RPT)])
        pltpu.sync_copy(ones_in, onesv)
        plsc.subcore_barrier()

        def make_body(e0, e1, table, acc, cc):
            def body(t, carry):
                off = wid * EPW + t * KC
                pltpu.sync_copy(e0.at[pl.ds(off, KC)], sidx)
                pltpu.sync_copy(e1.at[pl.ds(off, KC)], didx)
                pltpu.sync_copy(table.at[sidx], rows)
                pltpu.sync_copy(rows, acc.at[didx], add=True)
                pltpu.sync_copy(onesv, cc.at[didx], add=True)
                return carry
            return body

        lax.fori_loop(0, CPW, make_body(ea0, ea1, srca, accA, ccA), 0)
        lax.fori_loop(0, CPW, make_body(eb0, eb1, srcb, accB, ccB), 0)
        plsc.subcore_barrier()
        o0 = core * NPAD + r0
        pltpu.sync_copy(accA.at[pl.ds(r0, RPT)], aggrA.at[pl.ds(o0, RPT)])
        pltpu.sync_copy(accB.at[pl.ds(r0, RPT)], aggrB.at[pl.ds(o0, RPT)])
        pltpu.sync_copy(ccA.at[pl.ds(r0, RPT)], cntA.at[pl.ds(o0, RPT)])
        pltpu.sync_copy(ccB.at[pl.ds(r0, RPT)], cntB.at[pl.ds(o0, RPT)])

    return seg


def _seg2_kernel():
    @functools.partial(
        pl.kernel, mesh=_mesh(),
        compiler_params=pltpu.CompilerParams(**_SC_PARAMS),
        out_type=[_SDS((2 * NPAD, HID), jnp.float32)] * 2,
        scratch_types=[pltpu.VMEM((KC,), jnp.int32),
                       pltpu.VMEM((KC,), jnp.int32),
                       pltpu.VMEM((KC, HID), jnp.float32),
                       pltpu.VMEM_SHARED((NPAD, HID), jnp.float32),
                       pltpu.VMEM_SHARED((NPAD, HID), jnp.float32)],
    )
    def seg(srca, ea0, ea1, srcb, eb0, eb1, zrow,
            aggrA, aggrB,
            sidx, didx, rows, accA, accB):
        core = lax.axis_index("c")
        sub = lax.axis_index("s")
        wid = sub * 2 + core
        r0 = sub * RPT
        pltpu.sync_copy(zrow.at[pl.ds(r0, RPT)], accA.at[pl.ds(r0, RPT)])
        pltpu.sync_copy(zrow.at[pl.ds(r0, RPT)], accB.at[pl.ds(r0, RPT)])
        plsc.subcore_barrier()

        def make_body(e0, e1, table, acc):
            def body(t, carry):
                off = wid * EPW + t * KC
                pltpu.sync_copy(e0.at[pl.ds(off, KC)], sidx)
                pltpu.sync_copy(e1.at[pl.ds(off, KC)], didx)
                pltpu.sync_copy(table.at[sidx], rows)
                pltpu.sync_copy(rows, acc.at[didx], add=True)
                return carry
            return body

        lax.fori_loop(0, CPW, make_body(ea0, ea1, srca, accA), 0)
        lax.fori_loop(0, CPW, make_body(eb0, eb1, srcb, accB), 0)
        plsc.subcore_barrier()
        o0 = core * NPAD + r0
        pltpu.sync_copy(accA.at[pl.ds(r0, RPT)], aggrA.at[pl.ds(o0, RPT)])
        pltpu.sync_copy(accB.at[pl.ds(r0, RPT)], aggrB.at[pl.ds(o0, RPT)])

    return seg


def _head_kernel():
    @functools.partial(
        pl.kernel, mesh=_mesh(),
        compiler_params=pltpu.CompilerParams(**_SC_PARAMS),
        out_type=[_SDS((LPAD,), jnp.float32), _SDS((LPAD,), jnp.float32)],
        scratch_types=[pltpu.VMEM((KC,), jnp.int32),
                       pltpu.VMEM((KC,), jnp.int32),
                       pltpu.VMEM((KC, HID), jnp.float32),
                       pltpu.VMEM((KC, HID), jnp.float32),
                       pltpu.VMEM((KC,), jnp.float32)],
    )
    def head(h2g, h2d, la0, la1, lb0, lb1, predA, predB,
             ix, iy, ra, rb, dots):
        wid = lax.axis_index("s") * 2 + lax.axis_index("c")
        base = wid * LPW
        lanes = lax.broadcasted_iota(jnp.int32, (16,), 0)

        def make_body(l0, l1, tx, ty, out):
            def body(t, carry):
                off = base + t * KC
                pltpu.sync_copy(l0.at[pl.ds(off, KC)], ix)
                pltpu.sync_copy(l1.at[pl.ds(off, KC)], iy)
                pltpu.sync_copy(tx.at[ix], ra)
                pltpu.sync_copy(ty.at[iy], rb)
                for g in range(8):
                    rowids = lanes + (g * 16)
                    acc = jnp.zeros((16,), jnp.float32)
                    for dcol in range(HID):
                        cid = jnp.full((16,), dcol, jnp.int32)
                        a = plsc.load_gather(ra, [rowids, cid])
                        b = plsc.load_gather(rb, [rowids, cid])
                        acc = acc + a * b
                    dots[pl.ds(g * 16, 16)] = acc
                pltpu.sync_copy(dots, out.at[pl.ds(off, KC)])
                return carry
            return body

        lax.fori_loop(0, LCPW, make_body(la0, la1, h2g, h2d, predA), 0)
        lax.fori_loop(0, LCPW, make_body(lb0, lb1, h2d, h2g, predB), 0)

    return head


def kernel(x_gene, x_disease, edge_index_A, edge_index_B, label_A, label_B, params):
    f32, i32 = jnp.float32, jnp.int32
    xg = jnp.pad(x_gene, ((0, NPAD - N), (0, 0)))
    xd = jnp.pad(x_disease, ((0, NPAD - N), (0, 0)))
    pe = EPAD - E
    zpe = jnp.zeros((pe,), i32)
    dpe = jnp.full((pe,), N, i32)
    ea0 = jnp.concatenate([edge_index_A[0], zpe])
    ea1 = jnp.concatenate([edge_index_A[1], dpe])
    eb0 = jnp.concatenate([edge_index_B[0], zpe])
    eb1 = jnp.concatenate([edge_index_B[1], dpe])
    plp = LPAD - L
    zpl = jnp.zeros((plp,), i32)
    la0 = jnp.concatenate([label_A[0], zpl])
    la1 = jnp.concatenate([label_A[1], zpl])
    lb0 = jnp.concatenate([label_B[0], zpl])
    lb1 = jnp.concatenate([label_B[1], zpl])
    zrow = jnp.zeros((NPAD, HID), f32)
    ones_in = jnp.ones((KC, HID), f32)

    pA1, pB1, pA2, pB2 = params["A1"], params["B1"], params["A2"], params["B2"]
    _seg1, _seg2, _head = _seg1_kernel(), _seg2_kernel(), _head_kernel()
    tc_params = pltpu.CompilerParams(vmem_limit_bytes=100 * 1024 * 1024)

    srcA, dstA, srcB, dstB = pl.pallas_call(
        _t1_body, out_shape=[_SDS((NP4, 128), f32)] * 4,
        compiler_params=tc_params)(
        xg.reshape(NP4, 512), xd.reshape(NP4, 512),
        pA1["Wsrc"], pA1["bsrc"], pA1["Wdst"], pA1["bdst"], pA1["Wupd"], pA1["bupd"],
        pB1["Wsrc"], pB1["bsrc"], pB1["Wdst"], pB1["bdst"], pB1["Wupd"], pB1["bupd"])

    def _pk(a):
        return a.reshape(-1, 128)

    def _upk(a):
        return a.reshape(-1, HID)

    aggrA, cntA, aggrB, cntB = _seg1(_upk(srcA), ea0, ea1, _upk(srcB),
                                     eb0, eb1, zrow, ones_in)

    srcA2, dstA2, srcB2, dstB2 = pl.pallas_call(
        _t2_body, out_shape=[_SDS((NP4, 128), f32)] * 4,
        compiler_params=tc_params)(
        dstA, dstB, _pk(aggrA), _pk(cntA), _pk(aggrB), _pk(cntB),
        params["bn1_gene"]["gamma"], params["bn1_gene"]["beta"],
        params["bn1_dis"]["gamma"], params["bn1_dis"]["beta"],
        pA2["Wsrc"], pA2["bsrc"], pA2["Wdst"], pA2["bdst"], pA2["Wupd"], pA2["bupd"],
        pB2["Wsrc"], pB2["bsrc"], pB2["Wdst"], pB2["bdst"], pB2["Wupd"], pB2["bupd"])

    ag2A, ag2B = _seg2(_upk(srcA2), ea0, ea1, _upk(srcB2), eb0, eb1, zrow)

    h2g, h2d = pl.pallas_call(
        _t3_body, out_shape=[_SDS((NP4, 128), f32)] * 2,
        compiler_params=tc_params)(
        dstA2, dstB2, _pk(ag2A), _pk(ag2B), _pk(cntA), _pk(cntB),
        params["bn2_gene"]["gamma"], params["bn2_gene"]["beta"],
        params["bn2_dis"]["gamma"], params["bn2_dis"]["beta"])

    predA, predB = _head(_upk(h2g), _upk(h2d), la0, la1, lb0, lb1)
    return (predA[:L], predB[:L])


# speedup vs baseline: 4.9588x; 1.0719x over previous
import functools

import jax
import jax.numpy as jnp
from jax import lax
from jax.experimental import pallas as pl
from jax.experimental.pallas import tpu as pltpu
from jax.experimental.pallas import tpu_sc as plsc

HID = 32
N = 10000
NPAD = 10112
RPT = NPAD // 16
NP4 = NPAD // 4
NR4 = N // 4
E = 320000
KC = 128
NW = 32
CPW = 79
EPW = CPW * KC
EPAD = NW * EPW
L = 100000
LCPW = 25
LPW = LCPW * KC
LPAD = NW * LPW

_SDS = jax.ShapeDtypeStruct


def _mesh():
    return plsc.VectorSubcoreMesh(core_axis_name="c", subcore_axis_name="s")


def _dot(a, b):
    return jnp.dot(a, b, preferred_element_type=jnp.float32,
                   precision=lax.Precision.HIGHEST)


def _fold_src(ws, wu):
    return _dot(ws, wu[HID:, :])


def _fold_dst(wd, wu):
    return _dot(wd, wu[:HID, :])


def _fold_bias(bs, bd, wu, bu):
    return (_dot(bd.reshape(1, HID), wu[:HID, :])
            + _dot(bs.reshape(1, HID), wu[HID:, :]) + bu.reshape(1, HID))


def _bd4(w):
    z = jnp.zeros(w.shape, jnp.float32)
    return jnp.concatenate([
        jnp.concatenate([w, z, z, z], axis=1),
        jnp.concatenate([z, w, z, z], axis=1),
        jnp.concatenate([z, z, w, z], axis=1),
        jnp.concatenate([z, z, z, w], axis=1)], axis=0)


def _fold4(v):
    return v[0:HID] + v[HID:2 * HID] + v[2 * HID:3 * HID] + v[3 * HID:]


def _rep4(v):
    return jnp.concatenate([v, v, v, v])


def _bn_act_packed(h, gamma, beta, leaky):
    hv = h[0:NR4, :]
    mu = _rep4(_fold4(jnp.sum(hv, axis=0)) / N)
    e2 = _rep4(_fold4(jnp.sum(hv * hv, axis=0)) / N)
    var = e2 - mu * mu
    out = (_rep4(gamma)[None, :] * (h - mu[None, :])
           * lax.rsqrt(var[None, :] + 1e-5) + _rep4(beta)[None, :])
    if leaky:
        out = jnp.where(out >= 0, out, 0.01 * out)
    return out


def _t1_body(xg, xd, wsa, bsa, wda, bda, wua, bua,
             wsb, bsb, wdb, bdb, wub, bub,
             srcA, dstA, srcB, dstB):
    srcA[...] = _dot(xg[...], _bd4(_fold_src(wsa[...], wua[...])))
    dstA[...] = _dot(xd[...], _bd4(_fold_dst(wda[...], wua[...]))) + _rep4(
        _fold_bias(bsa[...], bda[...], wua[...], bua[...])[0])[None, :]
    srcB[...] = _dot(xd[...], _bd4(_fold_src(wsb[...], wub[...])))
    dstB[...] = _dot(xg[...], _bd4(_fold_dst(wdb[...], wub[...]))) + _rep4(
        _fold_bias(bsb[...], bdb[...], wub[...], bub[...])[0])[None, :]


def _t2_body(dstA, dstB, agA, cnA, agB, cnB,
             g1g, b1g, g1d, b1d,
             wsa, bsa, wda, bda, wua, bua,
             wsb, bsb, wdb, bdb, wub, bub,
             srcA2, dstA2, srcB2, dstB2):
    sumA = agA[0:NP4, :] + agA[NP4:, :]
    cdis = jnp.maximum(cnA[0:NP4, :] + cnA[NP4:, :], 1.0)
    h_dis = _bn_act_packed(dstA[...] + sumA / cdis, g1d[...], b1d[...], True)
    sumB = agB[0:NP4, :] + agB[NP4:, :]
    cgen = jnp.maximum(cnB[0:NP4, :] + cnB[NP4:, :], 1.0)
    h_gene = _bn_act_packed(dstB[...] + sumB / cgen, g1g[...], b1g[...], True)
    srcA2[...] = _dot(h_gene, _bd4(_fold_src(wsa[...], wua[...])))
    dstA2[...] = _dot(h_dis, _bd4(_fold_dst(wda[...], wua[...]))) + _rep4(
        _fold_bias(bsa[...], bda[...], wua[...], bua[...])[0])[None, :]
    srcB2[...] = _dot(h_dis, _bd4(_fold_src(wsb[...], wub[...])))
    dstB2[...] = _dot(h_gene, _bd4(_fold_dst(wdb[...], wub[...]))) + _rep4(
        _fold_bias(bsb[...], bdb[...], wub[...], bub[...])[0])[None, :]


def _t3_body(dstA2, dstB2, ag2A, ag2B, cnA, cnB,
             g2g, b2g, g2d, b2d, h2g, h2d):
    sumA = ag2A[0:NP4, :] + ag2A[NP4:, :]
    cdis = jnp.maximum(cnA[0:NP4, :] + cnA[NP4:, :], 1.0)
    h2d[...] = _bn_act_packed(dstA2[...] + sumA / cdis, g2d[...], b2d[...], False)
    sumB = ag2B[0:NP4, :] + ag2B[NP4:, :]
    cgen = jnp.maximum(cnB[0:NP4, :] + cnB[NP4:, :], 1.0)
    h2g[...] = _bn_act_packed(dstB2[...] + sumB / cgen, g2g[...], b2g[...], False)


_SC_PARAMS = dict(use_tc_tiling_on_sc=False, needs_layout_passes=False)


def _seg1_kernel():
    @functools.partial(
        pl.kernel, mesh=_mesh(),
        compiler_params=pltpu.CompilerParams(**_SC_PARAMS),
        out_type=[_SDS((2 * NPAD, HID), jnp.float32)] * 4,
        scratch_types=[pltpu.VMEM((KC,), jnp.int32),
                       pltpu.VMEM((KC,), jnp.int32),
                       pltpu.VMEM((KC, HID), jnp.float32),
                       pltpu.VMEM((KC, HID), jnp.float32),
                       pltpu.VMEM_SHARED((NPAD, HID), jnp.float32),
                       pltpu.VMEM_SHARED((NPAD, HID), jnp.float32),
                       pltpu.VMEM_SHARED((NPAD, HID), jnp.float32),
                       pltpu.VMEM_SHARED((NPAD, HID), jnp.float32)],
    )
    def seg(srca, ea0, ea1, srcb, eb0, eb1, zrow, ones_in,
            aggrA, cntA, aggrB, cntB,
            sidx, didx, rows, onesv, accA, accB, ccA, ccB):
        core = lax.axis_index("c")
        sub = lax.axis_index("s")
        wid = sub * 2 + core
        r0 = sub * RPT
        pltpu.sync_copy(zrow.at[pl.ds(r0, RPT)], accA.at[pl.ds(r0, RPT)])
        pltpu.sync_copy(zrow.at[pl.ds(r0, RPT)], accB.at[pl.ds(r0, RPT)])
        pltpu.sync_copy(zrow.at[pl.ds(r0, RPT)], ccA.at[pl.ds(r0, RPT)])
        pltpu.sync_copy(zrow.at[pl.ds(r0, RPT)], ccB.at[pl.ds(r0, RPT)])
        pltpu.sync_copy(ones_in, onesv)
        plsc.subcore_barrier()

        def make_body(e0, e1, table, acc, cc):
            def body(t, carry):
                off = wid * EPW + t * KC
                pltpu.sync_copy(e0.at[pl.ds(off, KC)], sidx)
                pltpu.sync_copy(e1.at[pl.ds(off, KC)], didx)
                pltpu.sync_copy(table.at[sidx], rows)
                pltpu.sync_copy(rows, acc.at[didx], add=True)
                pltpu.sync_copy(onesv, cc.at[didx], add=True)
                return carry
            return body

        lax.fori_loop(0, CPW, make_body(ea0, ea1, srca, accA, ccA), 0)
        lax.fori_loop(0, CPW, make_body(eb0, eb1, srcb, accB, ccB), 0)
        plsc.subcore_barrier()
        o0 = core * NPAD + r0
        pltpu.sync_copy(accA.at[pl.ds(r0, RPT)], aggrA.at[pl.ds(o0, RPT)])
        pltpu.sync_copy(accB.at[pl.ds(r0, RPT)], aggrB.at[pl.ds(o0, RPT)])
        pltpu.sync_copy(ccA.at[pl.ds(r0, RPT)], cntA.at[pl.ds(o0, RPT)])
        pltpu.sync_copy(ccB.at[pl.ds(r0, RPT)], cntB.at[pl.ds(o0, RPT)])

    return seg


def _seg2_kernel():
    @functools.partial(
        pl.kernel, mesh=_mesh(),
        compiler_params=pltpu.CompilerParams(**_SC_PARAMS),
        out_type=[_SDS((2 * NPAD, HID), jnp.float32)] * 2,
        scratch_types=[pltpu.VMEM((KC,), jnp.int32),
                       pltpu.VMEM((KC,), jnp.int32),
                       pltpu.VMEM((KC, HID), jnp.float32),
                       pltpu.VMEM_SHARED((NPAD, HID), jnp.float32),
                       pltpu.VMEM_SHARED((NPAD, HID), jnp.float32)],
    )
    def seg(srca, ea0, ea1, srcb, eb0, eb1, zrow,
            aggrA, aggrB,
            sidx, didx, rows, accA, accB):
        core = lax.axis_index("c")
        sub = lax.axis_index("s")
        wid = sub * 2 + core
        r0 = sub * RPT
        pltpu.sync_copy(zrow.at[pl.ds(r0, RPT)], accA.at[pl.ds(r0, RPT)])
        pltpu.sync_copy(zrow.at[pl.ds(r0, RPT)], accB.at[pl.ds(r0, RPT)])
        plsc.subcore_barrier()

        def make_body(e0, e1, table, acc):
            def body(t, carry):
                off = wid * EPW + t * KC
                pltpu.sync_copy(e0.at[pl.ds(off, KC)], sidx)
                pltpu.sync_copy(e1.at[pl.ds(off, KC)], didx)
                pltpu.sync_copy(table.at[sidx], rows)
                pltpu.sync_copy(rows, acc.at[didx], add=True)
                return carry
            return body

        lax.fori_loop(0, CPW, make_body(ea0, ea1, srca, accA), 0)
        lax.fori_loop(0, CPW, make_body(eb0, eb1, srcb, accB), 0)
        plsc.subcore_barrier()
        o0 = core * NPAD + r0
        pltpu.sync_copy(accA.at[pl.ds(r0, RPT)], aggrA.at[pl.ds(o0, RPT)])
        pltpu.sync_copy(accB.at[pl.ds(r0, RPT)], aggrB.at[pl.ds(o0, RPT)])

    return seg


def _head_kernel():
    @functools.partial(
        pl.kernel, mesh=_mesh(),
        compiler_params=pltpu.CompilerParams(**_SC_PARAMS),
        out_type=[_SDS((LPAD, HID), jnp.float32)] * 4,
        scratch_types=[pltpu.VMEM((KC,), jnp.int32),
                       pltpu.VMEM((KC,), jnp.int32),
                       pltpu.VMEM((KC, HID), jnp.float32),
                       pltpu.VMEM((KC, HID), jnp.float32)],
    )
    def head(h2g, h2d, la0, la1, lb0, lb1, gax, gay, gbx, gby,
             ix, iy, ra, rb):
        wid = lax.axis_index("s") * 2 + lax.axis_index("c")
        base = wid * LPW

        def make_body(l0, l1, tx, ty, ox, oy):
            def body(t, carry):
                off = base + t * KC
                pltpu.sync_copy(l0.at[pl.ds(off, KC)], ix)
                pltpu.sync_copy(l1.at[pl.ds(off, KC)], iy)
                pltpu.sync_copy(tx.at[ix], ra)
                pltpu.sync_copy(ty.at[iy], rb)
                pltpu.sync_copy(ra, ox.at[pl.ds(off, KC)])
                pltpu.sync_copy(rb, oy.at[pl.ds(off, KC)])
                return carry
            return body

        lax.fori_loop(0, LCPW, make_body(la0, la1, h2g, h2d, gax, gay), 0)
        lax.fori_loop(0, LCPW, make_body(lb0, lb1, h2d, h2g, gbx, gby), 0)

    return head


L4 = LPAD // 4


def _t4_body(gax, gay, gbx, gby, pa, pb):
    r = lax.broadcasted_iota(jnp.int32, (128, 4), 0)
    c = lax.broadcasted_iota(jnp.int32, (128, 4), 1)
    S = jnp.where(r // HID == c, 1.0, 0.0).astype(jnp.float32)
    pa[...] = _dot(gax[...] * gay[...], S)
    pb[...] = _dot(gbx[...] * gby[...], S)


def kernel(x_gene, x_disease, edge_index_A, edge_index_B, label_A, label_B, params):
    f32, i32 = jnp.float32, jnp.int32
    xg = jnp.pad(x_gene, ((0, NPAD - N), (0, 0)))
    xd = jnp.pad(x_disease, ((0, NPAD - N), (0, 0)))
    pe = EPAD - E
    zpe = jnp.zeros((pe,), i32)
    dpe = jnp.full((pe,), N, i32)
    ea0 = jnp.concatenate([edge_index_A[0], zpe])
    ea1 = jnp.concatenate([edge_index_A[1], dpe])
    eb0 = jnp.concatenate([edge_index_B[0], zpe])
    eb1 = jnp.concatenate([edge_index_B[1], dpe])
    plp = LPAD - L
    zpl = jnp.zeros((plp,), i32)
    la0 = jnp.concatenate([label_A[0], zpl])
    la1 = jnp.concatenate([label_A[1], zpl])
    lb0 = jnp.concatenate([label_B[0], zpl])
    lb1 = jnp.concatenate([label_B[1], zpl])
    zrow = jnp.zeros((NPAD, HID), f32)
    ones_in = jnp.ones((KC, HID), f32)

    pA1, pB1, pA2, pB2 = params["A1"], params["B1"], params["A2"], params["B2"]
    _seg1, _seg2, _head = _seg1_kernel(), _seg2_kernel(), _head_kernel()
    tc_params = pltpu.CompilerParams(vmem_limit_bytes=100 * 1024 * 1024)

    srcA, dstA, srcB, dstB = pl.pallas_call(
        _t1_body, out_shape=[_SDS((NP4, 128), f32)] * 4,
        compiler_params=tc_params)(
        xg.reshape(NP4, 512), xd.reshape(NP4, 512),
        pA1["Wsrc"], pA1["bsrc"], pA1["Wdst"], pA1["bdst"], pA1["Wupd"], pA1["bupd"],
        pB1["Wsrc"], pB1["bsrc"], pB1["Wdst"], pB1["bdst"], pB1["Wupd"], pB1["bupd"])

    def _pk(a):
        return a.reshape(-1, 128)

    def _upk(a):
        return a.reshape(-1, HID)

    aggrA, cntA, aggrB, cntB = _seg1(_upk(srcA), ea0, ea1, _upk(srcB),
                                     eb0, eb1, zrow, ones_in)

    srcA2, dstA2, srcB2, dstB2 = pl.pallas_call(
        _t2_body, out_shape=[_SDS((NP4, 128), f32)] * 4,
        compiler_params=tc_params)(
        dstA, dstB, _pk(aggrA), _pk(cntA), _pk(aggrB), _pk(cntB),
        params["bn1_gene"]["gamma"], params["bn1_gene"]["beta"],
        params["bn1_dis"]["gamma"], params["bn1_dis"]["beta"],
        pA2["Wsrc"], pA2["bsrc"], pA2["Wdst"], pA2["bdst"], pA2["Wupd"], pA2["bupd"],
        pB2["Wsrc"], pB2["bsrc"], pB2["Wdst"], pB2["bdst"], pB2["Wupd"], pB2["bupd"])

    ag2A, ag2B = _seg2(_upk(srcA2), ea0, ea1, _upk(srcB2), eb0, eb1, zrow)

    h2g, h2d = pl.pallas_call(
        _t3_body, out_shape=[_SDS((NP4, 128), f32)] * 2,
        compiler_params=tc_params)(
        dstA2, dstB2, _pk(ag2A), _pk(ag2B), _pk(cntA), _pk(cntB),
        params["bn2_gene"]["gamma"], params["bn2_gene"]["beta"],
        params["bn2_dis"]["gamma"], params["bn2_dis"]["beta"])

    gax, gay, gbx, gby = _head(_upk(h2g), _upk(h2d), la0, la1, lb0, lb1)
    nb = 8
    rb = L4 // nb
    pa, pb = pl.pallas_call(
        _t4_body, out_shape=[_SDS((L4, 4), f32)] * 2,
        grid=(nb,),
        in_specs=[pl.BlockSpec((rb, 128), lambda i: (i, 0))] * 4,
        out_specs=[pl.BlockSpec((rb, 4), lambda i: (i, 0))] * 2,
        compiler_params=tc_params)(_pk(gax), _pk(gay), _pk(gbx), _pk(gby))
    return (pa.reshape(LPAD)[:L], pb.reshape(LPAD)[:L])


# speedup vs baseline: 7.9026x; 1.5936x over previous
import functools

import jax
import jax.numpy as jnp
from jax import lax
from jax.experimental import pallas as pl
from jax.experimental.pallas import tpu as pltpu
from jax.experimental.pallas import tpu_sc as plsc

HID = 32
N = 10000
NPAD = 10112
RPT = NPAD // 16
NP4 = NPAD // 4
NR4 = N // 4
E = 320000
KC = 128
NW = 32
CPW = 80
EPW = CPW * KC
EPAD = NW * EPW
NBUF = 4
NSTEP = CPW // NBUF
L = 100000
LCPW = 25
LPW = LCPW * KC
LPAD = NW * LPW
HNBUF = 5
HSTEP = LCPW // HNBUF

_SDS = jax.ShapeDtypeStruct


def _mesh():
    return plsc.VectorSubcoreMesh(core_axis_name="c", subcore_axis_name="s")


def _dot(a, b):
    return jnp.dot(a, b, preferred_element_type=jnp.float32,
                   precision=lax.Precision.HIGHEST)


def _fold_src(ws, wu):
    return _dot(ws, wu[HID:, :])


def _fold_dst(wd, wu):
    return _dot(wd, wu[:HID, :])


def _fold_bias(bs, bd, wu, bu):
    return (_dot(bd.reshape(1, HID), wu[:HID, :])
            + _dot(bs.reshape(1, HID), wu[HID:, :]) + bu.reshape(1, HID))


def _bd4(w):
    z = jnp.zeros(w.shape, jnp.float32)
    return jnp.concatenate([
        jnp.concatenate([w, z, z, z], axis=1),
        jnp.concatenate([z, w, z, z], axis=1),
        jnp.concatenate([z, z, w, z], axis=1),
        jnp.concatenate([z, z, z, w], axis=1)], axis=0)


def _fold4(v):
    return v[0:HID] + v[HID:2 * HID] + v[2 * HID:3 * HID] + v[3 * HID:]


def _rep4(v):
    return jnp.concatenate([v, v, v, v])


def _bn_act_packed(h, gamma, beta, leaky):
    hv = h[0:NR4, :]
    mu = _rep4(_fold4(jnp.sum(hv, axis=0)) / N)
    e2 = _rep4(_fold4(jnp.sum(hv * hv, axis=0)) / N)
    var = e2 - mu * mu
    out = (_rep4(gamma)[None, :] * (h - mu[None, :])
           * lax.rsqrt(var[None, :] + 1e-5) + _rep4(beta)[None, :])
    if leaky:
        out = jnp.where(out >= 0, out, 0.01 * out)
    return out


def _t1_body(xg, xd, wsa, bsa, wda, bda, wua, bua,
             wsb, bsb, wdb, bdb, wub, bub,
             srcA, dstA, srcB, dstB):
    srcA[...] = _dot(xg[...], _bd4(_fold_src(wsa[...], wua[...])))
    dstA[...] = _dot(xd[...], _bd4(_fold_dst(wda[...], wua[...]))) + _rep4(
        _fold_bias(bsa[...], bda[...], wua[...], bua[...])[0])[None, :]
    srcB[...] = _dot(xd[...], _bd4(_fold_src(wsb[...], wub[...])))
    dstB[...] = _dot(xg[...], _bd4(_fold_dst(wdb[...], wub[...]))) + _rep4(
        _fold_bias(bsb[...], bdb[...], wub[...], bub[...])[0])[None, :]


def _t2_body(dstA, dstB, agA, cnA, agB, cnB,
             g1g, b1g, g1d, b1d,
             wsa, bsa, wda, bda, wua, bua,
             wsb, bsb, wdb, bdb, wub, bub,
             srcA2, dstA2, srcB2, dstB2):
    sumA = agA[0:NP4, :] + agA[NP4:, :]
    cdis = jnp.maximum(cnA[0:NP4, :] + cnA[NP4:, :], 1.0)
    h_dis = _bn_act_packed(dstA[...] + sumA / cdis, g1d[...], b1d[...], True)
    sumB = agB[0:NP4, :] + agB[NP4:, :]
    cgen = jnp.maximum(cnB[0:NP4, :] + cnB[NP4:, :], 1.0)
    h_gene = _bn_act_packed(dstB[...] + sumB / cgen, g1g[...], b1g[...], True)
    srcA2[...] = _dot(h_gene, _bd4(_fold_src(wsa[...], wua[...])))
    dstA2[...] = _dot(h_dis, _bd4(_fold_dst(wda[...], wua[...]))) + _rep4(
        _fold_bias(bsa[...], bda[...], wua[...], bua[...])[0])[None, :]
    srcB2[...] = _dot(h_dis, _bd4(_fold_src(wsb[...], wub[...])))
    dstB2[...] = _dot(h_gene, _bd4(_fold_dst(wdb[...], wub[...]))) + _rep4(
        _fold_bias(bsb[...], bdb[...], wub[...], bub[...])[0])[None, :]


def _t3_body(dstA2, dstB2, ag2A, ag2B, cnA, cnB,
             g2g, b2g, g2d, b2d, h2g, h2d):
    sumA = ag2A[0:NP4, :] + ag2A[NP4:, :]
    cdis = jnp.maximum(cnA[0:NP4, :] + cnA[NP4:, :], 1.0)
    h2d[...] = _bn_act_packed(dstA2[...] + sumA / cdis, g2d[...], b2d[...], False)
    sumB = ag2B[0:NP4, :] + ag2B[NP4:, :]
    cgen = jnp.maximum(cnB[0:NP4, :] + cnB[NP4:, :], 1.0)
    h2g[...] = _bn_act_packed(dstB2[...] + sumB / cgen, g2g[...], b2g[...], False)


_SC_PARAMS = dict(use_tc_tiling_on_sc=False, needs_layout_passes=False)


def _seg1_kernel():
    @functools.partial(
        pl.kernel, mesh=_mesh(),
        compiler_params=pltpu.CompilerParams(**_SC_PARAMS),
        out_type=[_SDS((2 * NPAD, HID), jnp.float32)] * 4,
        scratch_types=[pltpu.VMEM((CPW, KC), jnp.int32),
                       pltpu.VMEM((CPW, KC), jnp.int32),
                       pltpu.VMEM((KC, HID), jnp.float32)]
                      + [pltpu.VMEM((KC, HID), jnp.float32)] * NBUF
                      + [pltpu.SemaphoreType.DMA] * NBUF
                      + [pltpu.VMEM_SHARED((NPAD, HID), jnp.float32)] * 4,
    )
    def seg(srca, ea0, ea1, srcb, eb0, eb1, zrow, ones_in,
            aggrA, cntA, aggrB, cntB,
            sidx, didx, onesv, b0, b1, b2, b3, s0, s1, s2, s3,
            accA, accB, ccA, ccB):
        bufs = (b0, b1, b2, b3)
        sems = (s0, s1, s2, s3)
        core = lax.axis_index("c")
        sub = lax.axis_index("s")
        wid = sub * 2 + core
        r0 = sub * RPT
        pltpu.sync_copy(zrow.at[pl.ds(r0, RPT)], accA.at[pl.ds(r0, RPT)])
        pltpu.sync_copy(zrow.at[pl.ds(r0, RPT)], accB.at[pl.ds(r0, RPT)])
        pltpu.sync_copy(zrow.at[pl.ds(r0, RPT)], ccA.at[pl.ds(r0, RPT)])
        pltpu.sync_copy(zrow.at[pl.ds(r0, RPT)], ccB.at[pl.ds(r0, RPT)])
        pltpu.sync_copy(ones_in, onesv)
        plsc.subcore_barrier()

        def run(e0, e1, table, acc, cc):
            pltpu.sync_copy(e0.at[pl.ds(wid * CPW, CPW)], sidx)
            pltpu.sync_copy(e1.at[pl.ds(wid * CPW, CPW)], didx)
            for b in range(NBUF):
                pltpu.async_copy(table.at[sidx.at[b]], bufs[b], sems[b])

            def body(s, carry):
                for b in range(NBUF):
                    t = s * NBUF + b
                    pltpu.make_async_copy(
                        table.at[sidx.at[t]], bufs[b], sems[b]).wait()
                    pltpu.sync_copy(bufs[b], acc.at[didx.at[t]], add=True)
                    pltpu.sync_copy(onesv, cc.at[didx.at[t]], add=True)

                    @pl.when(t + NBUF < CPW)
                    def _():
                        pltpu.async_copy(
                            table.at[sidx.at[t + NBUF]], bufs[b], sems[b])
                return carry
            lax.fori_loop(0, NSTEP, body, 0)

        run(ea0, ea1, srca, accA, ccA)
        run(eb0, eb1, srcb, accB, ccB)
        plsc.subcore_barrier()
        o0 = core * NPAD + r0
        pltpu.sync_copy(accA.at[pl.ds(r0, RPT)], aggrA.at[pl.ds(o0, RPT)])
        pltpu.sync_copy(accB.at[pl.ds(r0, RPT)], aggrB.at[pl.ds(o0, RPT)])
        pltpu.sync_copy(ccA.at[pl.ds(r0, RPT)], cntA.at[pl.ds(o0, RPT)])
        pltpu.sync_copy(ccB.at[pl.ds(r0, RPT)], cntB.at[pl.ds(o0, RPT)])

    return seg


def _seg2_kernel():
    @functools.partial(
        pl.kernel, mesh=_mesh(),
        compiler_params=pltpu.CompilerParams(**_SC_PARAMS),
        out_type=[_SDS((2 * NPAD, HID), jnp.float32)] * 2,
        scratch_types=[pltpu.VMEM((CPW, KC), jnp.int32),
                       pltpu.VMEM((CPW, KC), jnp.int32)]
                      + [pltpu.VMEM((KC, HID), jnp.float32)] * NBUF
                      + [pltpu.SemaphoreType.DMA] * NBUF
                      + [pltpu.VMEM_SHARED((NPAD, HID), jnp.float32)] * 2,
    )
    def seg(srca, ea0, ea1, srcb, eb0, eb1, zrow,
            aggrA, aggrB,
            sidx, didx, b0, b1, b2, b3, s0, s1, s2, s3, accA, accB):
        bufs = (b0, b1, b2, b3)
        sems = (s0, s1, s2, s3)
        core = lax.axis_index("c")
        sub = lax.axis_index("s")
        wid = sub * 2 + core
        r0 = sub * RPT
        pltpu.sync_copy(zrow.at[pl.ds(r0, RPT)], accA.at[pl.ds(r0, RPT)])
        pltpu.sync_copy(zrow.at[pl.ds(r0, RPT)], accB.at[pl.ds(r0, RPT)])
        plsc.subcore_barrier()

        def run(e0, e1, table, acc):
            pltpu.sync_copy(e0.at[pl.ds(wid * CPW, CPW)], sidx)
            pltpu.sync_copy(e1.at[pl.ds(wid * CPW, CPW)], didx)
            for b in range(NBUF):
                pltpu.async_copy(table.at[sidx.at[b]], bufs[b], sems[b])

            def body(s, carry):
                for b in range(NBUF):
                    t = s * NBUF + b
                    pltpu.make_async_copy(
                        table.at[sidx.at[t]], bufs[b], sems[b]).wait()
                    pltpu.sync_copy(bufs[b], acc.at[didx.at[t]], add=True)

                    @pl.when(t + NBUF < CPW)
                    def _():
                        pltpu.async_copy(
                            table.at[sidx.at[t + NBUF]], bufs[b], sems[b])
                return carry
            lax.fori_loop(0, NSTEP, body, 0)

        run(ea0, ea1, srca, accA)
        run(eb0, eb1, srcb, accB)
        plsc.subcore_barrier()
        o0 = core * NPAD + r0
        pltpu.sync_copy(accA.at[pl.ds(r0, RPT)], aggrA.at[pl.ds(o0, RPT)])
        pltpu.sync_copy(accB.at[pl.ds(r0, RPT)], aggrB.at[pl.ds(o0, RPT)])

    return seg


def _head_kernel():
    @functools.partial(
        pl.kernel, mesh=_mesh(),
        compiler_params=pltpu.CompilerParams(**_SC_PARAMS),
        out_type=[_SDS((LPAD, HID), jnp.float32)] * 4,
        scratch_types=[pltpu.VMEM((LCPW, KC), jnp.int32),
                       pltpu.VMEM((LCPW, KC), jnp.int32)]
                      + [pltpu.VMEM((KC, HID), jnp.float32)] * (2 * HNBUF)
                      + [pltpu.SemaphoreType.DMA] * HNBUF,
    )
    def head(h2g, h2d, la0, la1, lb0, lb1, gax, gay, gbx, gby,
             ix, iy, a0, a1, a2, a3, a4, c0, c1, c2, c3, c4,
             s0, s1, s2, s3, s4):
        abufs = (a0, a1, a2, a3, a4)
        bbufs = (c0, c1, c2, c3, c4)
        sems = (s0, s1, s2, s3, s4)
        wid = lax.axis_index("s") * 2 + lax.axis_index("c")
        base = wid * LPW

        def run(l0, l1, tx, ty, ox, oy):
            pltpu.sync_copy(l0.at[pl.ds(wid * LCPW, LCPW)], ix)
            pltpu.sync_copy(l1.at[pl.ds(wid * LCPW, LCPW)], iy)
            for b in range(HNBUF):
                pltpu.async_copy(tx.at[ix.at[b]], abufs[b], sems[b])
                pltpu.async_copy(ty.at[iy.at[b]], bbufs[b], sems[b])

            def body(s, carry):
                for b in range(HNBUF):
                    t = s * HNBUF + b
                    off = base + t * KC
                    pltpu.make_async_copy(
                        tx.at[ix.at[t]], abufs[b], sems[b]).wait()
                    pltpu.make_async_copy(
                        ty.at[iy.at[t]], bbufs[b], sems[b]).wait()
                    pltpu.sync_copy(abufs[b], ox.at[pl.ds(off, KC)])
                    pltpu.sync_copy(bbufs[b], oy.at[pl.ds(off, KC)])

                    @pl.when(t + HNBUF < LCPW)
                    def _():
                        pltpu.async_copy(
                            tx.at[ix.at[t + HNBUF]], abufs[b], sems[b])
                        pltpu.async_copy(
                            ty.at[iy.at[t + HNBUF]], bbufs[b], sems[b])
                return carry
            lax.fori_loop(0, HSTEP, body, 0)

        run(la0, la1, h2g, h2d, gax, gay)
        run(lb0, lb1, h2d, h2g, gbx, gby)

    return head


L4 = LPAD // 4


def _t4_body(gax, gay, gbx, gby, pa, pb):
    r = lax.broadcasted_iota(jnp.int32, (128, 4), 0)
    c = lax.broadcasted_iota(jnp.int32, (128, 4), 1)
    S = jnp.where(r // HID == c, 1.0, 0.0).astype(jnp.float32)
    pa[...] = _dot(gax[...] * gay[...], S)
    pb[...] = _dot(gbx[...] * gby[...], S)


def kernel(x_gene, x_disease, edge_index_A, edge_index_B, label_A, label_B, params):
    f32, i32 = jnp.float32, jnp.int32
    xg = jnp.pad(x_gene, ((0, NPAD - N), (0, 0)))
    xd = jnp.pad(x_disease, ((0, NPAD - N), (0, 0)))
    pe = EPAD - E
    zpe = jnp.zeros((pe,), i32)
    dpe = jnp.full((pe,), N, i32)
    ea0 = jnp.concatenate([edge_index_A[0], zpe]).reshape(EPAD // KC, KC)
    ea1 = jnp.concatenate([edge_index_A[1], dpe]).reshape(EPAD // KC, KC)
    eb0 = jnp.concatenate([edge_index_B[0], zpe]).reshape(EPAD // KC, KC)
    eb1 = jnp.concatenate([edge_index_B[1], dpe]).reshape(EPAD // KC, KC)
    plp = LPAD - L
    zpl = jnp.zeros((plp,), i32)
    la0 = jnp.concatenate([label_A[0], zpl]).reshape(LPAD // KC, KC)
    la1 = jnp.concatenate([label_A[1], zpl]).reshape(LPAD // KC, KC)
    lb0 = jnp.concatenate([label_B[0], zpl]).reshape(LPAD // KC, KC)
    lb1 = jnp.concatenate([label_B[1], zpl]).reshape(LPAD // KC, KC)
    zrow = jnp.zeros((NPAD, HID), f32)
    ones_in = jnp.ones((KC, HID), f32)

    pA1, pB1, pA2, pB2 = params["A1"], params["B1"], params["A2"], params["B2"]
    _seg1, _seg2, _head = _seg1_kernel(), _seg2_kernel(), _head_kernel()
    tc_params = pltpu.CompilerParams(vmem_limit_bytes=100 * 1024 * 1024)

    srcA, dstA, srcB, dstB = pl.pallas_call(
        _t1_body, out_shape=[_SDS((NP4, 128), f32)] * 4,
        compiler_params=tc_params)(
        xg.reshape(NP4, 512), xd.reshape(NP4, 512),
        pA1["Wsrc"], pA1["bsrc"], pA1["Wdst"], pA1["bdst"], pA1["Wupd"], pA1["bupd"],
        pB1["Wsrc"], pB1["bsrc"], pB1["Wdst"], pB1["bdst"], pB1["Wupd"], pB1["bupd"])

    def _pk(a):
        return a.reshape(-1, 128)

    def _upk(a):
        return a.reshape(-1, HID)

    aggrA, cntA, aggrB, cntB = _seg1(_upk(srcA), ea0, ea1, _upk(srcB),
                                     eb0, eb1, zrow, ones_in)

    srcA2, dstA2, srcB2, dstB2 = pl.pallas_call(
        _t2_body, out_shape=[_SDS((NP4, 128), f32)] * 4,
        compiler_params=tc_params)(
        dstA, dstB, _pk(aggrA), _pk(cntA), _pk(aggrB), _pk(cntB),
        params["bn1_gene"]["gamma"], params["bn1_gene"]["beta"],
        params["bn1_dis"]["gamma"], params["bn1_dis"]["beta"],
        pA2["Wsrc"], pA2["bsrc"], pA2["Wdst"], pA2["bdst"], pA2["Wupd"], pA2["bupd"],
        pB2["Wsrc"], pB2["bsrc"], pB2["Wdst"], pB2["bdst"], pB2["Wupd"], pB2["bupd"])

    ag2A, ag2B = _seg2(_upk(srcA2), ea0, ea1, _upk(srcB2), eb0, eb1, zrow)

    h2g, h2d = pl.pallas_call(
        _t3_body, out_shape=[_SDS((NP4, 128), f32)] * 2,
        compiler_params=tc_params)(
        dstA2, dstB2, _pk(ag2A), _pk(ag2B), _pk(cntA), _pk(cntB),
        params["bn2_gene"]["gamma"], params["bn2_gene"]["beta"],
        params["bn2_dis"]["gamma"], params["bn2_dis"]["beta"])

    gax, gay, gbx, gby = _head(_upk(h2g), _upk(h2d), la0, la1, lb0, lb1)
    nb = 8
    rb = L4 // nb
    pa, pb = pl.pallas_call(
        _t4_body, out_shape=[_SDS((L4, 4), f32)] * 2,
        grid=(nb,),
        in_specs=[pl.BlockSpec((rb, 128), lambda i: (i, 0))] * 4,
        out_specs=[pl.BlockSpec((rb, 4), lambda i: (i, 0))] * 2,
        compiler_params=tc_params)(_pk(gax), _pk(gay), _pk(gbx), _pk(gby))
    return (pa.reshape(LPAD)[:L], pb.reshape(LPAD)[:L])


# speedup vs baseline: 7.9138x; 1.0014x over previous
import functools

import jax
import jax.numpy as jnp
from jax import lax
from jax.experimental import pallas as pl
from jax.experimental.pallas import tpu as pltpu
from jax.experimental.pallas import tpu_sc as plsc

HID = 32
N = 10000
NPAD = 10112
RPT = NPAD // 16
NP4 = NPAD // 4
NR4 = N // 4
E = 320000
KC = 128
NW = 32
CPW = 80
EPW = CPW * KC
EPAD = NW * EPW
NBUF1 = 5
NBUF2 = 8
L = 100000
LCPW = 25
LPW = LCPW * KC
LPAD = NW * LPW
HNBUF = 5
HSTEP = LCPW // HNBUF

_SDS = jax.ShapeDtypeStruct


def _mesh():
    return plsc.VectorSubcoreMesh(core_axis_name="c", subcore_axis_name="s")


def _dot(a, b):
    return jnp.dot(a, b, preferred_element_type=jnp.float32,
                   precision=lax.Precision.HIGHEST)


def _fold_src(ws, wu):
    return _dot(ws, wu[HID:, :])


def _fold_dst(wd, wu):
    return _dot(wd, wu[:HID, :])


def _fold_bias(bs, bd, wu, bu):
    return (_dot(bd.reshape(1, HID), wu[:HID, :])
            + _dot(bs.reshape(1, HID), wu[HID:, :]) + bu.reshape(1, HID))


def _bd4(w):
    z = jnp.zeros(w.shape, jnp.float32)
    return jnp.concatenate([
        jnp.concatenate([w, z, z, z], axis=1),
        jnp.concatenate([z, w, z, z], axis=1),
        jnp.concatenate([z, z, w, z], axis=1),
        jnp.concatenate([z, z, z, w], axis=1)], axis=0)


def _fold4(v):
    return v[0:HID] + v[HID:2 * HID] + v[2 * HID:3 * HID] + v[3 * HID:]


def _rep4(v):
    return jnp.concatenate([v, v, v, v])


def _bn_act_packed(h, gamma, beta, leaky):
    hv = h[0:NR4, :]
    mu = _rep4(_fold4(jnp.sum(hv, axis=0)) / N)
    e2 = _rep4(_fold4(jnp.sum(hv * hv, axis=0)) / N)
    var = e2 - mu * mu
    out = (_rep4(gamma)[None, :] * (h - mu[None, :])
           * lax.rsqrt(var[None, :] + 1e-5) + _rep4(beta)[None, :])
    if leaky:
        out = jnp.where(out >= 0, out, 0.01 * out)
    return out


def _t1_body(xg, xd, wsa, bsa, wda, bda, wua, bua,
             wsb, bsb, wdb, bdb, wub, bub,
             srcA, dstA, srcB, dstB):
    srcA[...] = _dot(xg[...], _bd4(_fold_src(wsa[...], wua[...])))
    dstA[...] = _dot(xd[...], _bd4(_fold_dst(wda[...], wua[...]))) + _rep4(
        _fold_bias(bsa[...], bda[...], wua[...], bua[...])[0])[None, :]
    srcB[...] = _dot(xd[...], _bd4(_fold_src(wsb[...], wub[...])))
    dstB[...] = _dot(xg[...], _bd4(_fold_dst(wdb[...], wub[...]))) + _rep4(
        _fold_bias(bsb[...], bdb[...], wub[...], bub[...])[0])[None, :]


def _t2_body(dstA, dstB, agA, cnA, agB, cnB,
             g1g, b1g, g1d, b1d,
             wsa, bsa, wda, bda, wua, bua,
             wsb, bsb, wdb, bdb, wub, bub,
             srcA2, dstA2, srcB2, dstB2):
    sumA = agA[0:NP4, :] + agA[NP4:, :]
    cdis = jnp.maximum(cnA[0:NP4, :] + cnA[NP4:, :], 1.0)
    h_dis = _bn_act_packed(dstA[...] + sumA / cdis, g1d[...], b1d[...], True)
    sumB = agB[0:NP4, :] + agB[NP4:, :]
    cgen = jnp.maximum(cnB[0:NP4, :] + cnB[NP4:, :], 1.0)
    h_gene = _bn_act_packed(dstB[...] + sumB / cgen, g1g[...], b1g[...], True)
    srcA2[...] = _dot(h_gene, _bd4(_fold_src(wsa[...], wua[...])))
    dstA2[...] = _dot(h_dis, _bd4(_fold_dst(wda[...], wua[...]))) + _rep4(
        _fold_bias(bsa[...], bda[...], wua[...], bua[...])[0])[None, :]
    srcB2[...] = _dot(h_dis, _bd4(_fold_src(wsb[...], wub[...])))
    dstB2[...] = _dot(h_gene, _bd4(_fold_dst(wdb[...], wub[...]))) + _rep4(
        _fold_bias(bsb[...], bdb[...], wub[...], bub[...])[0])[None, :]


def _t3_body(dstA2, dstB2, ag2A, ag2B, cnA, cnB,
             g2g, b2g, g2d, b2d, h2g, h2d):
    sumA = ag2A[0:NP4, :] + ag2A[NP4:, :]
    cdis = jnp.maximum(cnA[0:NP4, :] + cnA[NP4:, :], 1.0)
    h2d[...] = _bn_act_packed(dstA2[...] + sumA / cdis, g2d[...], b2d[...], False)
    sumB = ag2B[0:NP4, :] + ag2B[NP4:, :]
    cgen = jnp.maximum(cnB[0:NP4, :] + cnB[NP4:, :], 1.0)
    h2g[...] = _bn_act_packed(dstB2[...] + sumB / cgen, g2g[...], b2g[...], False)


_SC_PARAMS = dict(use_tc_tiling_on_sc=False, needs_layout_passes=False)


def _seg_kernel(nbuf, counts):
    nout = 4 if counts else 2
    nin = 8 if counts else 7
    scratch = ([pltpu.VMEM((CPW, KC), jnp.int32)] * 2
               + ([pltpu.VMEM((KC, HID), jnp.float32)] if counts else [])
               + [pltpu.VMEM((KC, HID), jnp.float32)] * nbuf
               + [pltpu.SemaphoreType.DMA] * nbuf
               + [pltpu.VMEM_SHARED((NPAD, HID), jnp.float32)] * nout)
    nstep = CPW // nbuf

    @functools.partial(
        pl.kernel, mesh=_mesh(),
        compiler_params=pltpu.CompilerParams(**_SC_PARAMS),
        out_type=[_SDS((2 * NPAD, HID), jnp.float32)] * nout,
        scratch_types=scratch,
    )
    def seg(*a):
        if counts:
            (srca, ea0, ea1, srcb, eb0, eb1, zrow, ones_in,
             aggrA, cntA, aggrB, cntB) = a[:nin + nout]
        else:
            (srca, ea0, ea1, srcb, eb0, eb1, zrow,
             aggrA, aggrB) = a[:nin + nout]
        sc = list(a[nin + nout:])
        sidx, didx = sc[0], sc[1]
        p = 2
        if counts:
            onesv = sc[p]
            p += 1
        bufs = sc[p:p + nbuf]
        sems = sc[p + nbuf:p + 2 * nbuf]
        accs = sc[p + 2 * nbuf:]
        if counts:
            accA, accB, ccA, ccB = accs
        else:
            accA, accB = accs
        core = lax.axis_index("c")
        sub = lax.axis_index("s")
        wid = sub * 2 + core
        r0 = sub * RPT
        for acc in accs:
            pltpu.sync_copy(zrow.at[pl.ds(r0, RPT)], acc.at[pl.ds(r0, RPT)])
        if counts:
            pltpu.sync_copy(ones_in, onesv)
        plsc.subcore_barrier()

        def run(e0, e1, table, acc, cc):
            pltpu.sync_copy(e0.at[pl.ds(wid * CPW, CPW)], sidx)
            pltpu.sync_copy(e1.at[pl.ds(wid * CPW, CPW)], didx)
            for b in range(nbuf):
                pltpu.async_copy(table.at[sidx.at[b]], bufs[b], sems[b])

            def body(s, carry):
                for b in range(nbuf):
                    t = s * nbuf + b
                    pltpu.make_async_copy(
                        table.at[sidx.at[t]], bufs[b], sems[b]).wait()
                    pltpu.sync_copy(bufs[b], acc.at[didx.at[t]], add=True)
                    if counts:
                        pltpu.sync_copy(onesv, cc.at[didx.at[t]], add=True)

                    @pl.when(t + nbuf < CPW)
                    def _():
                        pltpu.async_copy(
                            table.at[sidx.at[t + nbuf]], bufs[b], sems[b])
                return carry
            lax.fori_loop(0, nstep, body, 0)

        run(ea0, ea1, srca, accA, ccA if counts else None)
        run(eb0, eb1, srcb, accB, ccB if counts else None)
        plsc.subcore_barrier()
        o0 = core * NPAD + r0
        pltpu.sync_copy(accA.at[pl.ds(r0, RPT)], aggrA.at[pl.ds(o0, RPT)])
        pltpu.sync_copy(accB.at[pl.ds(r0, RPT)], aggrB.at[pl.ds(o0, RPT)])
        if counts:
            pltpu.sync_copy(ccA.at[pl.ds(r0, RPT)], cntA.at[pl.ds(o0, RPT)])
            pltpu.sync_copy(ccB.at[pl.ds(r0, RPT)], cntB.at[pl.ds(o0, RPT)])

    return seg


def _head_kernel():
    @functools.partial(
        pl.kernel, mesh=_mesh(),
        compiler_params=pltpu.CompilerParams(**_SC_PARAMS),
        out_type=[_SDS((LPAD, HID), jnp.float32)] * 4,
        scratch_types=[pltpu.VMEM((LCPW, KC), jnp.int32),
                       pltpu.VMEM((LCPW, KC), jnp.int32)]
                      + [pltpu.VMEM((KC, HID), jnp.float32)] * (2 * HNBUF)
                      + [pltpu.SemaphoreType.DMA] * HNBUF,
    )
    def head(h2g, h2d, la0, la1, lb0, lb1, gax, gay, gbx, gby,
             ix, iy, a0, a1, a2, a3, a4, c0, c1, c2, c3, c4,
             s0, s1, s2, s3, s4):
        abufs = (a0, a1, a2, a3, a4)
        bbufs = (c0, c1, c2, c3, c4)
        sems = (s0, s1, s2, s3, s4)
        wid = lax.axis_index("s") * 2 + lax.axis_index("c")
        base = wid * LPW

        def run(l0, l1, tx, ty, ox, oy):
            pltpu.sync_copy(l0.at[pl.ds(wid * LCPW, LCPW)], ix)
            pltpu.sync_copy(l1.at[pl.ds(wid * LCPW, LCPW)], iy)
            for b in range(HNBUF):
                pltpu.async_copy(tx.at[ix.at[b]], abufs[b], sems[b])
                pltpu.async_copy(ty.at[iy.at[b]], bbufs[b], sems[b])

            def body(s, carry):
                for b in range(HNBUF):
                    t = s * HNBUF + b
                    off = base + t * KC
                    pltpu.make_async_copy(
                        tx.at[ix.at[t]], abufs[b], sems[b]).wait()
                    pltpu.make_async_copy(
                        ty.at[iy.at[t]], bbufs[b], sems[b]).wait()
                    pltpu.sync_copy(abufs[b], ox.at[pl.ds(off, KC)])
                    pltpu.sync_copy(bbufs[b], oy.at[pl.ds(off, KC)])

                    @pl.when(t + HNBUF < LCPW)
                    def _():
                        pltpu.async_copy(
                            tx.at[ix.at[t + HNBUF]], abufs[b], sems[b])
                        pltpu.async_copy(
                            ty.at[iy.at[t + HNBUF]], bbufs[b], sems[b])
                return carry
            lax.fori_loop(0, HSTEP, body, 0)

        run(la0, la1, h2g, h2d, gax, gay)
        run(lb0, lb1, h2d, h2g, gbx, gby)

    return head


L4 = LPAD // 4


def _t4_body(gax, gay, gbx, gby, pa, pb):
    r = lax.broadcasted_iota(jnp.int32, (128, 4), 0)
    c = lax.broadcasted_iota(jnp.int32, (128, 4), 1)
    S = jnp.where(r // HID == c, 1.0, 0.0).astype(jnp.float32)
    pa[...] = _dot(gax[...] * gay[...], S)
    pb[...] = _dot(gbx[...] * gby[...], S)


def kernel(x_gene, x_disease, edge_index_A, edge_index_B, label_A, label_B, params):
    f32, i32 = jnp.float32, jnp.int32
    xg = jnp.pad(x_gene, ((0, NPAD - N), (0, 0)))
    xd = jnp.pad(x_disease, ((0, NPAD - N), (0, 0)))
    pe = EPAD - E
    zpe = jnp.zeros((pe,), i32)
    dpe = jnp.full((pe,), N, i32)
    ea0 = jnp.concatenate([edge_index_A[0], zpe]).reshape(EPAD // KC, KC)
    ea1 = jnp.concatenate([edge_index_A[1], dpe]).reshape(EPAD // KC, KC)
    eb0 = jnp.concatenate([edge_index_B[0], zpe]).reshape(EPAD // KC, KC)
    eb1 = jnp.concatenate([edge_index_B[1], dpe]).reshape(EPAD // KC, KC)
    plp = LPAD - L
    zpl = jnp.zeros((plp,), i32)
    la0 = jnp.concatenate([label_A[0], zpl]).reshape(LPAD // KC, KC)
    la1 = jnp.concatenate([label_A[1], zpl]).reshape(LPAD // KC, KC)
    lb0 = jnp.concatenate([label_B[0], zpl]).reshape(LPAD // KC, KC)
    lb1 = jnp.concatenate([label_B[1], zpl]).reshape(LPAD // KC, KC)
    zrow = jnp.zeros((NPAD, HID), f32)
    ones_in = jnp.ones((KC, HID), f32)

    pA1, pB1, pA2, pB2 = params["A1"], params["B1"], params["A2"], params["B2"]
    _seg1 = _seg_kernel(NBUF1, True)
    _seg2 = _seg_kernel(NBUF2, False)
    _head = _head_kernel()
    tc_params = pltpu.CompilerParams(vmem_limit_bytes=100 * 1024 * 1024)

    srcA, dstA, srcB, dstB = pl.pallas_call(
        _t1_body, out_shape=[_SDS((NP4, 128), f32)] * 4,
        compiler_params=tc_params)(
        xg.reshape(NP4, 512), xd.reshape(NP4, 512),
        pA1["Wsrc"], pA1["bsrc"], pA1["Wdst"], pA1["bdst"], pA1["Wupd"], pA1["bupd"],
        pB1["Wsrc"], pB1["bsrc"], pB1["Wdst"], pB1["bdst"], pB1["Wupd"], pB1["bupd"])

    def _pk(a):
        return a.reshape(-1, 128)

    def _upk(a):
        return a.reshape(-1, HID)

    aggrA, cntA, aggrB, cntB = _seg1(_upk(srcA), ea0, ea1, _upk(srcB),
                                     eb0, eb1, zrow, ones_in)

    srcA2, dstA2, srcB2, dstB2 = pl.pallas_call(
        _t2_body, out_shape=[_SDS((NP4, 128), f32)] * 4,
        compiler_params=tc_params)(
        dstA, dstB, _pk(aggrA), _pk(cntA), _pk(aggrB), _pk(cntB),
        params["bn1_gene"]["gamma"], params["bn1_gene"]["beta"],
        params["bn1_dis"]["gamma"], params["bn1_dis"]["beta"],
        pA2["Wsrc"], pA2["bsrc"], pA2["Wdst"], pA2["bdst"], pA2["Wupd"], pA2["bupd"],
        pB2["Wsrc"], pB2["bsrc"], pB2["Wdst"], pB2["bdst"], pB2["Wupd"], pB2["bupd"])

    ag2A, ag2B = _seg2(_upk(srcA2), ea0, ea1, _upk(srcB2), eb0, eb1, zrow)

    h2g, h2d = pl.pallas_call(
        _t3_body, out_shape=[_SDS((NP4, 128), f32)] * 2,
        compiler_params=tc_params)(
        dstA2, dstB2, _pk(ag2A), _pk(ag2B), _pk(cntA), _pk(cntB),
        params["bn2_gene"]["gamma"], params["bn2_gene"]["beta"],
        params["bn2_dis"]["gamma"], params["bn2_dis"]["beta"])

    gax, gay, gbx, gby = _head(_upk(h2g), _upk(h2d), la0, la1, lb0, lb1)
    nb = 8
    rb = L4 // nb
    pa, pb = pl.pallas_call(
        _t4_body, out_shape=[_SDS((L4, 4), f32)] * 2,
        grid=(nb,),
        in_specs=[pl.BlockSpec((rb, 128), lambda i: (i, 0))] * 4,
        out_specs=[pl.BlockSpec((rb, 4), lambda i: (i, 0))] * 2,
        compiler_params=tc_params)(_pk(gax), _pk(gay), _pk(gbx), _pk(gby))
    return (pa.reshape(LPAD)[:L], pb.reshape(LPAD)[:L])
